# baseline (speedup 1.0000x reference)
import functools

import jax
import jax.numpy as jnp
from jax import lax
from jax.experimental import pallas as pl
from jax.experimental.pallas import tpu as pltpu
from jax.experimental.pallas import tpu_sc as plsc

_N = 10000
_D = 128
_E = 320000

_NC = 2
_NS = 16
_NW = _NC * _NS
_EPW = _E // _NW
_CH = 80
_NCH = _EPW // _CH
_SB = 5
_NSB = _NCH // _SB
_ASB = 25
_ANSB = _NCH // _ASB

_RZ = 624
_RZ_LAST = _N - 15 * _RZ

_mesh = plsc.VectorSubcoreMesh(core_axis_name="c", subcore_axis_name="s")


@functools.partial(
    pl.kernel,
    mesh=_mesh,
    out_type=jax.ShapeDtypeStruct((_NC * _N,), jnp.float32),
    scratch_types=[
        pltpu.VMEM((_NSB, _SB, _CH), jnp.int32),
        pltpu.VMEM((_NSB, _SB, _CH), jnp.float32),
        pltpu.VMEM((_RZ_LAST,), jnp.float32),
        pltpu.VMEM_SHARED((_N,), jnp.float32),
        pltpu.SemaphoreType.DMA,
    ],
)
def _deg_kernel(dst_hbm, w_hbm, out_hbm, didx_v, w_v, zb_v, deg_sh, dsem):
    c = lax.axis_index("c")
    s = lax.axis_index("s")
    wid = c * _NS + s

    def zfill(i, carry):
        zb_v[pl.ds(i * 16, 16)] = jnp.zeros((16,), jnp.float32)
        return carry

    lax.fori_loop(0, _RZ_LAST // 16, zfill, 0)

    @pl.when(s < 15)
    def _():
        pltpu.sync_copy(zb_v.at[pl.ds(0, _RZ)], deg_sh.at[pl.ds(s * _RZ, _RZ)])

    @pl.when(s == 15)
    def _():
        pltpu.sync_copy(zb_v, deg_sh.at[pl.ds(15 * _RZ, _RZ_LAST)])

    pltpu.sync_copy(dst_hbm.at[wid], didx_v)
    pltpu.sync_copy(w_hbm.at[wid], w_v)
    plsc.subcore_barrier()

    def dchunk(i, carry):
        @pl.when(i >= 8)
        def _():
            pltpu.make_async_copy(w_v.at[0, 0], deg_sh.at[didx_v.at[0, 0]],
                                  dsem).wait()

        pltpu.async_copy(w_v.at[i // _SB, i % _SB],
                         deg_sh.at[didx_v.at[i // _SB, i % _SB]], dsem,
                         add=True)
        return carry

    lax.fori_loop(0, _NCH, dchunk, 0)

    def ddrain(i, carry):
        pltpu.make_async_copy(w_v.at[0, 0], deg_sh.at[didx_v.at[0, 0]],
                              dsem).wait()
        return carry

    lax.fori_loop(0, 8, ddrain, 0)
    plsc.subcore_barrier()

    @pl.when(s < 15)
    def _():
        pltpu.sync_copy(deg_sh.at[pl.ds(s * _RZ, _RZ)], zb_v.at[pl.ds(0, _RZ)])
        pltpu.sync_copy(zb_v.at[pl.ds(0, _RZ)],
                        out_hbm.at[pl.ds(c * _N + s * _RZ, _RZ)])

    @pl.when(s == 15)
    def _():
        pltpu.sync_copy(deg_sh.at[pl.ds(15 * _RZ, _RZ_LAST)], zb_v)
        pltpu.sync_copy(zb_v, out_hbm.at[pl.ds(c * _N + 15 * _RZ, _RZ_LAST)])


@functools.partial(
    pl.kernel,
    mesh=_mesh,
    out_type=jax.ShapeDtypeStruct((_NC, _N, _D), jnp.float32),
    scratch_types=[
        pltpu.VMEM((_ASB, _CH), jnp.int32),
        pltpu.VMEM((_ASB, _CH), jnp.int32),
        pltpu.VMEM((_ASB, _CH), jnp.float32),
        pltpu.VMEM((_CH, _D), jnp.float32),
        pltpu.VMEM_SHARED((_N, _D), jnp.float32),
        pltpu.SemaphoreType.DMA,
    ],
)
def _agg_kernel(h_hbm, src_hbm, dst_hbm, w_hbm, out_hbm,
                sidx_v, didx_v, w_v, rows_v, acc_sh, gsem):
    c = lax.axis_index("c")
    s = lax.axis_index("s")
    wid = c * _NS + s

    def zfill(i, carry):
        for g in range(_D // 16):
            rows_v[i, pl.ds(g * 16, 16)] = jnp.zeros((16,), jnp.float32)
        return carry

    lax.fori_loop(0, _CH, zfill, 0)

    zrows = rows_v
    n_full = _RZ // _CH
    for j in range(n_full):
        pltpu.sync_copy(zrows, acc_sh.at[pl.ds(s * _RZ + j * _CH, _CH)])
    rem = _RZ - n_full * _CH
    pltpu.sync_copy(zrows.at[pl.ds(0, rem)],
                    acc_sh.at[pl.ds(s * _RZ + n_full * _CH, rem)])

    @pl.when(s == 15)
    def _():
        pltpu.sync_copy(zrows.at[pl.ds(0, _RZ_LAST - _RZ)],
                        acc_sh.at[pl.ds(15 * _RZ + _RZ, _RZ_LAST - _RZ)])

    plsc.subcore_barrier()

    _dn = lax.GatherDimensionNumbers(
        offset_dims=(), collapsed_slice_dims=(0,), start_index_map=(0,))

    def sub_block(sb, sbcarry):
        pltpu.sync_copy(src_hbm.at[wid, sb], sidx_v)
        pltpu.sync_copy(dst_hbm.at[wid, sb], didx_v)
        pltpu.sync_copy(w_hbm.at[wid, sb], w_v)

        def chunk(j, carry):
            pltpu.async_copy(h_hbm.at[sidx_v.at[j]], rows_v,
                             gsem).wait()

            def grp(g, gcarry):
                w16 = w_v[j, pl.ds(g * 16, 16)]
                base = g * 16
                for l in range(16):
                    wb = lax.gather(
                        w16, jnp.full((16, 1), l, jnp.int32), _dn,
                        slice_sizes=(1,),
                        mode=lax.GatherScatterMode.PROMISE_IN_BOUNDS)
                    k = base + l
                    for cg in range(_D // 16):
                        rows_v[k, pl.ds(cg * 16, 16)] = (
                            rows_v[k, pl.ds(cg * 16, 16)] * wb)
                return gcarry

            for g in range(_CH // 16):
                grp(g, 0)
            pltpu.sync_copy(rows_v, acc_sh.at[didx_v.at[j]], add=True)
            return carry

        return lax.fori_loop(0, _ASB, chunk, sbcarry)

    lax.fori_loop(0, _ANSB, sub_block, 0)
    plsc.subcore_barrier()

    for j in range(n_full):
        pltpu.sync_copy(acc_sh.at[pl.ds(s * _RZ + j * _CH, _CH)], zrows)
        pltpu.sync_copy(zrows,
                        out_hbm.at[c, pl.ds(s * _RZ + j * _CH, _CH)])
    pltpu.sync_copy(acc_sh.at[pl.ds(s * _RZ + n_full * _CH, rem)],
                    zrows.at[pl.ds(0, rem)])
    pltpu.sync_copy(zrows.at[pl.ds(0, rem)],
                    out_hbm.at[c, pl.ds(s * _RZ + n_full * _CH, rem)])

    @pl.when(s == 15)
    def _():
        pltpu.sync_copy(acc_sh.at[pl.ds(15 * _RZ + _RZ, _RZ_LAST - _RZ)],
                        zrows.at[pl.ds(0, _RZ_LAST - _RZ)])
        pltpu.sync_copy(zrows.at[pl.ds(0, _RZ_LAST - _RZ)],
                        out_hbm.at[c, pl.ds(15 * _RZ + _RZ, _RZ_LAST - _RZ)])


_BN = 1000


def _tc_scale_matmul_body(deg_ref, x_ref, w_ref, h_ref):
    degp = deg_ref[...]
    dis = lax.rsqrt(degp[0] + degp[1] + 1.0)
    h = jnp.dot(x_ref[...], w_ref[...], preferred_element_type=jnp.float32)
    h_ref[...] = h * dis


def _tc_combine_matmul_body(deg_ref, agg_ref, h_ref, b_ref, w_ref, o_ref):
    degp = deg_ref[...]
    dis = lax.rsqrt(degp[0] + degp[1] + 1.0)
    agg = agg_ref[...]
    z = (agg[0] + agg[1] + h_ref[...]) * dis + b_ref[...]
    r = jnp.maximum(z, 0.0)
    h = jnp.dot(r, w_ref[...], preferred_element_type=jnp.float32)
    o_ref[...] = h * dis


def _tc_combine_body(deg_ref, agg_ref, h_ref, b_ref, o_ref):
    degp = deg_ref[...]
    dis = lax.rsqrt(degp[0] + degp[1] + 1.0)
    agg = agg_ref[...]
    o_ref[...] = (agg[0] + agg[1] + h_ref[...]) * dis + b_ref[...]


def kernel(x, edge_index, edge_weight, W0, b0, W1, b1):
    dstD = edge_index[1].reshape(_NW, _NSB, _SB, _CH)
    ewD = edge_weight.reshape(_NW, _NSB, _SB, _CH)
    src = edge_index[0].reshape(_NW, _ANSB, _ASB, _CH)
    dst = edge_index[1].reshape(_NW, _ANSB, _ASB, _CH)
    ew = edge_weight.reshape(_NW, _ANSB, _ASB, _CH)

    deg_p = _deg_kernel(dstD, ewD).reshape(_NC, _N, 1)

    h0 = pl.pallas_call(
        _tc_scale_matmul_body,
        grid=(_N // _BN,),
        in_specs=[
            pl.BlockSpec((2, _BN, 1), lambda i: (0, i, 0)),
            pl.BlockSpec((_BN, _D), lambda i: (i, 0)),
            pl.BlockSpec((_D, _D), lambda i: (0, 0)),
        ],
        out_specs=pl.BlockSpec((_BN, _D), lambda i: (i, 0)),
        out_shape=jax.ShapeDtypeStruct((_N, _D), jnp.float32),
    )(deg_p, x, W0)

    agg0 = _agg_kernel(h0, src, dst, ew)

    h1 = pl.pallas_call(
        _tc_combine_matmul_body,
        grid=(_N // _BN,),
        in_specs=[
            pl.BlockSpec((2, _BN, 1), lambda i: (0, i, 0)),
            pl.BlockSpec((2, _BN, _D), lambda i: (0, i, 0)),
            pl.BlockSpec((_BN, _D), lambda i: (i, 0)),
            pl.BlockSpec((1, _D), lambda i: (0, 0)),
            pl.BlockSpec((_D, _D), lambda i: (0, 0)),
        ],
        out_specs=pl.BlockSpec((_BN, _D), lambda i: (i, 0)),
        out_shape=jax.ShapeDtypeStruct((_N, _D), jnp.float32),
    )(deg_p, agg0, h0, b0.reshape(1, _D), W1)

    agg1 = _agg_kernel(h1, src, dst, ew)

    out = pl.pallas_call(
        _tc_combine_body,
        grid=(_N // _BN,),
        in_specs=[
            pl.BlockSpec((2, _BN, 1), lambda i: (0, i, 0)),
            pl.BlockSpec((2, _BN, _D), lambda i: (0, i, 0)),
            pl.BlockSpec((_BN, _D), lambda i: (i, 0)),
            pl.BlockSpec((1, _D), lambda i: (0, 0)),
        ],
        out_specs=pl.BlockSpec((_BN, _D), lambda i: (i, 0)),
        out_shape=jax.ShapeDtypeStruct((_N, _D), jnp.float32),
    )(deg_p, agg1, h1, b1.reshape(1, _D))

    return out


# speedup vs baseline: 1.0029x; 1.0029x over previous
import functools

import jax
import jax.numpy as jnp
from jax import lax
from jax.experimental import pallas as pl
from jax.experimental.pallas import tpu as pltpu
from jax.experimental.pallas import tpu_sc as plsc

_N = 10000
_D = 128
_E = 320000

_NC = 2
_NS = 16
_NW = _NC * _NS
_EPW = _E // _NW
_CH = 80
_NCH = _EPW // _CH
_SB = 5
_NSB = _NCH // _SB
_ASB = 25
_ANSB = _NCH // _ASB

_RZ = 624
_RZ_LAST = _N - 15 * _RZ

_mesh = plsc.VectorSubcoreMesh(core_axis_name="c", subcore_axis_name="s")


@functools.partial(
    pl.kernel,
    mesh=_mesh,
    out_type=jax.ShapeDtypeStruct((_NC * _N,), jnp.float32),
    scratch_types=[
        pltpu.VMEM((_NSB, _SB, _CH), jnp.int32),
        pltpu.VMEM((_NSB, _SB, _CH), jnp.float32),
        pltpu.VMEM((_RZ_LAST,), jnp.float32),
        pltpu.VMEM_SHARED((_N,), jnp.float32),
        pltpu.SemaphoreType.DMA,
    ],
)
def _deg_kernel(dst_hbm, w_hbm, out_hbm, didx_v, w_v, zb_v, deg_sh, dsem):
    c = lax.axis_index("c")
    s = lax.axis_index("s")
    wid = c * _NS + s

    def zfill(i, carry):
        zb_v[pl.ds(i * 16, 16)] = jnp.zeros((16,), jnp.float32)
        return carry

    lax.fori_loop(0, _RZ_LAST // 16, zfill, 0)

    @pl.when(s < 15)
    def _():
        pltpu.sync_copy(zb_v.at[pl.ds(0, _RZ)], deg_sh.at[pl.ds(s * _RZ, _RZ)])

    @pl.when(s == 15)
    def _():
        pltpu.sync_copy(zb_v, deg_sh.at[pl.ds(15 * _RZ, _RZ_LAST)])

    pltpu.sync_copy(dst_hbm.at[wid], didx_v)
    pltpu.sync_copy(w_hbm.at[wid], w_v)
    plsc.subcore_barrier()

    def dchunk(i, carry):
        @pl.when(i >= 8)
        def _():
            pltpu.make_async_copy(w_v.at[0, 0], deg_sh.at[didx_v.at[0, 0]],
                                  dsem).wait()

        pltpu.async_copy(w_v.at[i // _SB, i % _SB],
                         deg_sh.at[didx_v.at[i // _SB, i % _SB]], dsem,
                         add=True)
        return carry

    lax.fori_loop(0, _NCH, dchunk, 0)

    def ddrain(i, carry):
        pltpu.make_async_copy(w_v.at[0, 0], deg_sh.at[didx_v.at[0, 0]],
                              dsem).wait()
        return carry

    lax.fori_loop(0, 8, ddrain, 0)
    plsc.subcore_barrier()

    @pl.when(s < 15)
    def _():
        pltpu.sync_copy(deg_sh.at[pl.ds(s * _RZ, _RZ)], zb_v.at[pl.ds(0, _RZ)])
        pltpu.sync_copy(zb_v.at[pl.ds(0, _RZ)],
                        out_hbm.at[pl.ds(c * _N + s * _RZ, _RZ)])

    @pl.when(s == 15)
    def _():
        pltpu.sync_copy(deg_sh.at[pl.ds(15 * _RZ, _RZ_LAST)], zb_v)
        pltpu.sync_copy(zb_v, out_hbm.at[pl.ds(c * _N + 15 * _RZ, _RZ_LAST)])


@functools.partial(
    pl.kernel,
    mesh=_mesh,
    out_type=jax.ShapeDtypeStruct((_NC, _N, _D), jnp.float32),
    scratch_types=[
        pltpu.VMEM((_ASB, _CH), jnp.int32),
        pltpu.VMEM((_ASB, _CH), jnp.int32),
        pltpu.VMEM((_ASB, _CH), jnp.float32),
        pltpu.VMEM((_CH, _D), jnp.float32),
        pltpu.VMEM_SHARED((_N, _D), jnp.float32),
        pltpu.SemaphoreType.DMA,
    ],
)
def _agg_kernel(h_hbm, src_hbm, dst_hbm, w_hbm, out_hbm,
                sidx_v, didx_v, w_v, rows_v, acc_sh, gsem):
    c = lax.axis_index("c")
    s = lax.axis_index("s")
    wid = c * _NS + s

    def zfill(i, carry):
        for g in range(_D // 16):
            rows_v[i, pl.ds(g * 16, 16)] = jnp.zeros((16,), jnp.float32)
        return carry

    lax.fori_loop(0, _CH, zfill, 0)

    zrows = rows_v
    n_full = _RZ // _CH
    for j in range(n_full):
        pltpu.sync_copy(zrows, acc_sh.at[pl.ds(s * _RZ + j * _CH, _CH)])
    rem = _RZ - n_full * _CH
    pltpu.sync_copy(zrows.at[pl.ds(0, rem)],
                    acc_sh.at[pl.ds(s * _RZ + n_full * _CH, rem)])

    @pl.when(s == 15)
    def _():
        pltpu.sync_copy(zrows.at[pl.ds(0, _RZ_LAST - _RZ)],
                        acc_sh.at[pl.ds(15 * _RZ + _RZ, _RZ_LAST - _RZ)])

    plsc.subcore_barrier()

    _dn = lax.GatherDimensionNumbers(
        offset_dims=(), collapsed_slice_dims=(0,), start_index_map=(0,))

    def sub_block(sb, sbcarry):
        pltpu.sync_copy(src_hbm.at[wid, sb], sidx_v)
        pltpu.sync_copy(dst_hbm.at[wid, sb], didx_v)
        pltpu.sync_copy(w_hbm.at[wid, sb], w_v)

        def chunk(j, carry):
            pltpu.async_copy(h_hbm.at[sidx_v.at[j]], rows_v,
                             gsem).wait()

            def grp(g, gcarry):
                w16 = w_v[j, pl.ds(g * 16, 16)]
                base = g * 16
                for l in range(16):
                    wb = lax.gather(
                        w16, jnp.full((16, 1), l, jnp.int32), _dn,
                        slice_sizes=(1,),
                        mode=lax.GatherScatterMode.PROMISE_IN_BOUNDS)
                    k = base + l
                    for cg in range(_D // 16):
                        rows_v[k, pl.ds(cg * 16, 16)] = (
                            rows_v[k, pl.ds(cg * 16, 16)] * wb)
                return gcarry

            lax.fori_loop(0, _CH // 16, grp, 0)
            pltpu.sync_copy(rows_v, acc_sh.at[didx_v.at[j]], add=True)
            return carry

        return lax.fori_loop(0, _ASB, chunk, sbcarry)

    lax.fori_loop(0, _ANSB, sub_block, 0)
    plsc.subcore_barrier()

    for j in range(n_full):
        pltpu.sync_copy(acc_sh.at[pl.ds(s * _RZ + j * _CH, _CH)], zrows)
        pltpu.sync_copy(zrows,
                        out_hbm.at[c, pl.ds(s * _RZ + j * _CH, _CH)])
    pltpu.sync_copy(acc_sh.at[pl.ds(s * _RZ + n_full * _CH, rem)],
                    zrows.at[pl.ds(0, rem)])
    pltpu.sync_copy(zrows.at[pl.ds(0, rem)],
                    out_hbm.at[c, pl.ds(s * _RZ + n_full * _CH, rem)])

    @pl.when(s == 15)
    def _():
        pltpu.sync_copy(acc_sh.at[pl.ds(15 * _RZ + _RZ, _RZ_LAST - _RZ)],
                        zrows.at[pl.ds(0, _RZ_LAST - _RZ)])
        pltpu.sync_copy(zrows.at[pl.ds(0, _RZ_LAST - _RZ)],
                        out_hbm.at[c, pl.ds(15 * _RZ + _RZ, _RZ_LAST - _RZ)])


_BN = 1000


def _tc_scale_matmul_body(deg_ref, x_ref, w_ref, h_ref):
    degp = deg_ref[...]
    dis = lax.rsqrt(degp[0] + degp[1] + 1.0)
    h = jnp.dot(x_ref[...], w_ref[...], preferred_element_type=jnp.float32)
    h_ref[...] = h * dis


def _tc_combine_matmul_body(deg_ref, agg_ref, h_ref, b_ref, w_ref, o_ref):
    degp = deg_ref[...]
    dis = lax.rsqrt(degp[0] + degp[1] + 1.0)
    agg = agg_ref[...]
    z = (agg[0] + agg[1] + h_ref[...]) * dis + b_ref[...]
    r = jnp.maximum(z, 0.0)
    h = jnp.dot(r, w_ref[...], preferred_element_type=jnp.float32)
    o_ref[...] = h * dis


def _tc_combine_body(deg_ref, agg_ref, h_ref, b_ref, o_ref):
    degp = deg_ref[...]
    dis = lax.rsqrt(degp[0] + degp[1] + 1.0)
    agg = agg_ref[...]
    o_ref[...] = (agg[0] + agg[1] + h_ref[...]) * dis + b_ref[...]


def kernel(x, edge_index, edge_weight, W0, b0, W1, b1):
    dstD = edge_index[1].reshape(_NW, _NSB, _SB, _CH)
    ewD = edge_weight.reshape(_NW, _NSB, _SB, _CH)
    src = edge_index[0].reshape(_NW, _ANSB, _ASB, _CH)
    dst = edge_index[1].reshape(_NW, _ANSB, _ASB, _CH)
    ew = edge_weight.reshape(_NW, _ANSB, _ASB, _CH)

    deg_p = _deg_kernel(dstD, ewD).reshape(_NC, _N, 1)

    h0 = pl.pallas_call(
        _tc_scale_matmul_body,
        grid=(_N // _BN,),
        in_specs=[
            pl.BlockSpec((2, _BN, 1), lambda i: (0, i, 0)),
            pl.BlockSpec((_BN, _D), lambda i: (i, 0)),
            pl.BlockSpec((_D, _D), lambda i: (0, 0)),
        ],
        out_specs=pl.BlockSpec((_BN, _D), lambda i: (i, 0)),
        out_shape=jax.ShapeDtypeStruct((_N, _D), jnp.float32),
    )(deg_p, x, W0)

    agg0 = _agg_kernel(h0, src, dst, ew)

    h1 = pl.pallas_call(
        _tc_combine_matmul_body,
        grid=(_N // _BN,),
        in_specs=[
            pl.BlockSpec((2, _BN, 1), lambda i: (0, i, 0)),
            pl.BlockSpec((2, _BN, _D), lambda i: (0, i, 0)),
            pl.BlockSpec((_BN, _D), lambda i: (i, 0)),
            pl.BlockSpec((1, _D), lambda i: (0, 0)),
            pl.BlockSpec((_D, _D), lambda i: (0, 0)),
        ],
        out_specs=pl.BlockSpec((_BN, _D), lambda i: (i, 0)),
        out_shape=jax.ShapeDtypeStruct((_N, _D), jnp.float32),
    )(deg_p, agg0, h0, b0.reshape(1, _D), W1)

    agg1 = _agg_kernel(h1, src, dst, ew)

    out = pl.pallas_call(
        _tc_combine_body,
        grid=(_N // _BN,),
        in_specs=[
            pl.BlockSpec((2, _BN, 1), lambda i: (0, i, 0)),
            pl.BlockSpec((2, _BN, _D), lambda i: (0, i, 0)),
            pl.BlockSpec((_BN, _D), lambda i: (i, 0)),
            pl.BlockSpec((1, _D), lambda i: (0, 0)),
        ],
        out_specs=pl.BlockSpec((_BN, _D), lambda i: (i, 0)),
        out_shape=jax.ShapeDtypeStruct((_N, _D), jnp.float32),
    )(deg_p, agg1, h1, b1.reshape(1, _D))

    return out
